# baseline (speedup 1.0000x reference)
import functools
import jax
import jax.numpy as jnp
from jax.experimental import pallas as pl
from jax.experimental.pallas import tpu as pltpu

NLAYER = 20
TE = 640
EPS = 1e-5


def _sweep_loop(E, body, init):
    nt = E // TE
    return jax.lax.fori_loop(0, nt, body, init, unroll=2)


def _lane_slice(i):
    return pl.ds(pl.multiple_of(i * TE, 128), TE)


def _stats_to_scale(s, q, g, b, E):
    m = s * (1.0 / E)
    var = jnp.maximum(q * (1.0 / E) - m * m, 0.0)
    a = g * jax.lax.rsqrt(var + EPS)
    c = b - m * a
    return a, c


def _layer_body(g_ref, h_ref,
                ew1_ref, eb1_ref, eg1_ref, ec1_ref,
                ew2_ref, eb2_ref, eg2_ref, ec2_ref,
                ew3_ref, eb3_ref,
                vw1_ref, vb1_ref, vg1_ref, vc1_ref,
                vw2_ref, vb2_ref, vg2_ref, vc2_ref,
                vw3_ref, vb3_ref,
                hn_ref, msg_ref):
    E = g_ref.shape[1]
    f32 = jnp.float32
    dot = lambda a, b: jax.lax.dot_general(
        a, b, (((1,), (0,)), ((), ())), preferred_element_type=f32)

    W1g = ew1_ref[:, 0:6]
    W1h = ew1_ref[:, 6:9]
    b1 = eb1_ref[:]
    W2 = ew2_ref[:]
    b2 = eb2_ref[:]
    W3 = ew3_ref[:]
    b3 = eb3_ref[:]
    V1m = vw1_ref[:, 0:3]
    V1h = vw1_ref[:, 3:6]
    vb1 = vb1_ref[:]
    V2 = vw2_ref[:]
    vb2 = vb2_ref[:]
    V3 = vw3_ref[:]
    vb3 = vb3_ref[:]

    def x1_at(i):
        sl = _lane_slice(i)
        G = g_ref[:, sl]
        Ht = h_ref[:, sl]
        return G, dot(W1g, G[0:6]) + dot(W1h, Ht) + b1

    def s1(i, carry):
        s, q = carry
        _, X = x1_at(i)
        return s + jnp.sum(X, axis=1, keepdims=True), \
               q + jnp.sum(X * X, axis=1, keepdims=True)
    z20 = jnp.zeros((20, 1), f32)
    s, q = _sweep_loop(E, s1, (z20, z20))
    a1, c1 = _stats_to_scale(s, q, eg1_ref[:], ec1_ref[:], E)

    def s2(i, carry):
        s, q = carry
        _, X = x1_at(i)
        T1 = jnp.maximum(a1 * X + c1, 0.0)
        X2 = dot(W2, T1) + b2
        return s + jnp.sum(X2, axis=1, keepdims=True), \
               q + jnp.sum(X2 * X2, axis=1, keepdims=True)
    s, q = _sweep_loop(E, s2, (z20, z20))
    a2, c2 = _stats_to_scale(s, q, eg2_ref[:], ec2_ref[:], E)

    def s3(i, carry):
        s, q = carry
        sl = _lane_slice(i)
        G, X = x1_at(i)
        T1 = jnp.maximum(a1 * X + c1, 0.0)
        T2 = jnp.maximum(a2 * (dot(W2, T1) + b2) + c2, 0.0)
        Hn = dot(W3, T2) + b3
        hn_ref[:, sl] = Hn
        Y1 = dot(V1m, G[0:3]) + dot(V1h, Hn) + vb1
        return s + jnp.sum(Y1, axis=1, keepdims=True), \
               q + jnp.sum(Y1 * Y1, axis=1, keepdims=True)
    s, q = _sweep_loop(E, s3, (z20, z20))
    ay1, cy1 = _stats_to_scale(s, q, vg1_ref[:], vc1_ref[:], E)

    def y1_at(i):
        sl = _lane_slice(i)
        Mi = g_ref[0:3, sl]
        Hn = hn_ref[:, sl]
        return dot(V1m, Mi) + dot(V1h, Hn) + vb1

    def s4(i, carry):
        s, q = carry
        U1 = jnp.maximum(ay1 * y1_at(i) + cy1, 0.0)
        Y2 = dot(V2, U1) + vb2
        return s + jnp.sum(Y2, axis=1, keepdims=True), \
               q + jnp.sum(Y2 * Y2, axis=1, keepdims=True)
    s, q = _sweep_loop(E, s4, (z20, z20))
    ay2, cy2 = _stats_to_scale(s, q, vg2_ref[:], vc2_ref[:], E)

    def s5(i, _):
        sl = _lane_slice(i)
        U1 = jnp.maximum(ay1 * y1_at(i) + cy1, 0.0)
        U2 = jnp.maximum(ay2 * (dot(V2, U1) + vb2) + cy2, 0.0)
        msg_ref[:, sl] = dot(V3, U2) + vb3
        return 0
    _sweep_loop(E, s5, 0)


def _final_body(g_ref, h_ref,
                ew1_ref, eb1_ref, eg1_ref, ec1_ref,
                ew2_ref, eb2_ref, eg2_ref, ec2_ref,
                ew3_ref, eb3_ref,
                ow1_ref, ob1_ref, og1_ref, oc1_ref,
                ow2_ref, ob2_ref, og2_ref, oc2_ref,
                ow3_ref, ob3_ref,
                out_ref, hn_ref):
    E = g_ref.shape[1]
    f32 = jnp.float32
    dot = lambda a, b: jax.lax.dot_general(
        a, b, (((1,), (0,)), ((), ())), preferred_element_type=f32)

    W1g = ew1_ref[:, 0:6]
    W1h = ew1_ref[:, 6:9]
    b1 = eb1_ref[:]
    W2 = ew2_ref[:]
    b2 = eb2_ref[:]
    W3 = ew3_ref[:]
    b3 = eb3_ref[:]
    O1 = ow1_ref[:]
    ob1 = ob1_ref[:]
    O2 = ow2_ref[:]
    ob2 = ob2_ref[:]
    O3 = ow3_ref[:]
    ob3 = ob3_ref[:]

    def x1_at(i):
        sl = _lane_slice(i)
        return dot(W1g, g_ref[:, sl][0:6]) + dot(W1h, h_ref[:, sl]) + b1

    def s1(i, carry):
        s, q = carry
        X = x1_at(i)
        return s + jnp.sum(X, axis=1, keepdims=True), \
               q + jnp.sum(X * X, axis=1, keepdims=True)
    z20 = jnp.zeros((20, 1), f32)
    s, q = _sweep_loop(E, s1, (z20, z20))
    a1, c1 = _stats_to_scale(s, q, eg1_ref[:], ec1_ref[:], E)

    def s2(i, carry):
        s, q = carry
        T1 = jnp.maximum(a1 * x1_at(i) + c1, 0.0)
        X2 = dot(W2, T1) + b2
        return s + jnp.sum(X2, axis=1, keepdims=True), \
               q + jnp.sum(X2 * X2, axis=1, keepdims=True)
    s, q = _sweep_loop(E, s2, (z20, z20))
    a2, c2 = _stats_to_scale(s, q, eg2_ref[:], ec2_ref[:], E)

    def s3(i, carry):
        s, q = carry
        sl = _lane_slice(i)
        T1 = jnp.maximum(a1 * x1_at(i) + c1, 0.0)
        T2 = jnp.maximum(a2 * (dot(W2, T1) + b2) + c2, 0.0)
        Hn = dot(W3, T2) + b3
        hn_ref[:, sl] = Hn
        P1 = dot(O1, Hn) + ob1
        return s + jnp.sum(P1, axis=1, keepdims=True), \
               q + jnp.sum(P1 * P1, axis=1, keepdims=True)
    z3 = jnp.zeros((3, 1), f32)
    s, q = _sweep_loop(E, s3, (z3, z3))
    ao1, co1 = _stats_to_scale(s, q, og1_ref[:], oc1_ref[:], E)

    def p1_at(i):
        sl = _lane_slice(i)
        return dot(O1, hn_ref[:, sl]) + ob1

    def s4(i, carry):
        s, q = carry
        R1 = jnp.maximum(ao1 * p1_at(i) + co1, 0.0)
        P2 = dot(O2, R1) + ob2
        return s + jnp.sum(P2, axis=1, keepdims=True), \
               q + jnp.sum(P2 * P2, axis=1, keepdims=True)
    s, q = _sweep_loop(E, s4, (z3, z3))
    ao2, co2 = _stats_to_scale(s, q, og2_ref[:], oc2_ref[:], E)

    def s5(i, _):
        sl = _lane_slice(i)
        R1 = jnp.maximum(ao1 * p1_at(i) + co1, 0.0)
        R2 = jnp.maximum(ao2 * (dot(O2, R1) + ob2) + co2, 0.0)
        out_ref[:, sl] = dot(O3, R2) + ob3
        return 0
    _sweep_loop(E, s5, 0)


def _layer_weights(p):
    c2 = lambda a: a.reshape(-1, 1)
    return (p['e_l1_W'], c2(p['e_l1_b']), c2(p['e_bn1_g']), c2(p['e_bn1_b']),
            p['e_l2_W'], c2(p['e_l2_b']), c2(p['e_bn2_g']), c2(p['e_bn2_b']),
            p['e_l3_W'], c2(p['e_l3_b']),
            p['v_l1_W'], c2(p['v_l1_b']), c2(p['v_bn1_g']), c2(p['v_bn1_b']),
            p['v_l2_W'], c2(p['v_l2_b']), c2(p['v_bn2_g']), c2(p['v_bn2_b']),
            p['v_l3_W'], c2(p['v_l3_b']))


def _o_weights(p):
    c2 = lambda a: a.reshape(-1, 1)
    return (p['o_l1_W'], c2(p['o_l1_b']), c2(p['o_bn1_g']), c2(p['o_bn1_b']),
            p['o_l2_W'], c2(p['o_l2_b']), c2(p['o_bn2_g']), c2(p['o_bn2_b']),
            p['o_l3_W'], c2(p['o_l3_b']))


def kernel(M, H, edge_index, params):
    N = M.shape[0]
    E = H.shape[0]
    src = edge_index[0]
    dst = edge_index[1]
    p = params
    f32 = jnp.float32

    layer_call = pl.pallas_call(
        _layer_body,
        out_shape=(jax.ShapeDtypeStruct((3, E), f32),
                   jax.ShapeDtypeStruct((3, E), f32)),
    )
    final_call = pl.pallas_call(
        _final_body,
        out_shape=jax.ShapeDtypeStruct((2, E), f32),
        scratch_shapes=[pltpu.VMEM((3, E), f32)],
    )

    lw = _layer_weights(p)
    ow = _o_weights(p)

    Ht = H.T
    for _ in range(NLAYER - 1):
        G = jnp.concatenate([M[dst], M[src]], axis=1).T
        Ht, MsgT = layer_call(G, Ht, *lw)
        M = jax.ops.segment_sum(MsgT.T, dst, num_segments=N)
    G = jnp.concatenate([M[dst], M[src]], axis=1).T
    out_t = final_call(G, Ht, *lw[:10], *ow)
    return out_t.T


# speedup vs baseline: 2.1755x; 2.1755x over previous
import functools
import jax
import jax.numpy as jnp
from jax import lax
from jax.experimental import pallas as pl
from jax.experimental.pallas import tpu as pltpu
from jax.experimental.pallas import tpu_sc as plsc

NLAYER = 20
N_NODES = 10000
N_EDGES = 160000
_NC = 2
_NS = 16
_CHUNK = N_EDGES // (_NC * _NS)
_CPAD = ((_CHUNK + 15) // 16) * 16
_NGRP = _CPAD // 16


def _gather_body(m_hbm, dst_hbm, src_hbm, mi_hbm, mj_hbm,
                 mloc, di, si, gi, gj):
    wid = lax.axis_index("s") * _NC + lax.axis_index("c")
    base = wid * _CHUNK

    pltpu.sync_copy(m_hbm, mloc)
    zeros16 = jnp.zeros((16,), jnp.int32)
    di[pl.ds(_CPAD - 16, 16)] = zeros16
    si[pl.ds(_CPAD - 16, 16)] = zeros16
    pltpu.sync_copy(dst_hbm.at[pl.ds(base, _CHUNK)], di.at[pl.ds(0, _CHUNK)])
    pltpu.sync_copy(src_hbm.at[pl.ds(base, _CHUNK)], si.at[pl.ds(0, _CHUNK)])

    iota16 = lax.broadcasted_iota(jnp.int32, (16,), 0)

    def grp(g, carry):
        off = g * 16
        idxd = di[pl.ds(off, 16)]
        idxs = si[pl.ds(off, 16)]
        pos = (off + iota16) * 3
        fd = idxd * 3
        fs = idxs * 3
        for c in range(3):
            vd = plsc.load_gather(mloc, [fd + c])
            plsc.store_scatter(gi, [pos + c], vd)
            vs = plsc.load_gather(mloc, [fs + c])
            plsc.store_scatter(gj, [pos + c], vs)
        return carry

    lax.fori_loop(0, _NGRP, grp, 0)

    pltpu.sync_copy(gi.at[pl.ds(0, _CHUNK * 3)],
                    mi_hbm.at[pl.ds(base * 3, _CHUNK * 3)])
    pltpu.sync_copy(gj.at[pl.ds(0, _CHUNK * 3)],
                    mj_hbm.at[pl.ds(base * 3, _CHUNK * 3)])


def _make_gather():
    mesh = plsc.VectorSubcoreMesh(core_axis_name="c", subcore_axis_name="s")
    f32 = jnp.float32
    return functools.partial(
        pl.kernel,
        mesh=mesh,
        compiler_params=pltpu.CompilerParams(needs_layout_passes=False),
        out_type=(jax.ShapeDtypeStruct((N_EDGES * 3,), f32),
                  jax.ShapeDtypeStruct((N_EDGES * 3,), f32)),
        scratch_types=[
            pltpu.VMEM((N_NODES * 3,), f32),
            pltpu.VMEM((_CPAD,), jnp.int32),
            pltpu.VMEM((_CPAD,), jnp.int32),
            pltpu.VMEM((_CPAD * 3,), f32),
            pltpu.VMEM((_CPAD * 3,), f32),
        ],
    )(_gather_body)


def _bn(x, g, b, eps=1e-5):
    mean = jnp.mean(x, axis=0)
    var = jnp.var(x, axis=0)
    return g * (x - mean) / jnp.sqrt(var + eps) + b


def _mlp3(x, p, pre):
    x = x @ p[pre + '_l1_W'].T + p[pre + '_l1_b']
    x = _bn(x, p[pre + '_bn1_g'], p[pre + '_bn1_b'])
    x = jax.nn.relu(x)
    x = x @ p[pre + '_l2_W'].T + p[pre + '_l2_b']
    x = _bn(x, p[pre + '_bn2_g'], p[pre + '_bn2_b'])
    x = jax.nn.relu(x)
    x = x @ p[pre + '_l3_W'].T + p[pre + '_l3_b']
    return x


def _mlp_o_body(h_ref, w1_ref, b1_ref, g1_ref, bb1_ref, w2_ref, b2_ref,
                g2_ref, bb2_ref, w3_ref, b3_ref, o_ref):
    x = h_ref[:]
    eps = 1e-5
    x = jnp.dot(w1_ref[:], x, preferred_element_type=jnp.float32) + b1_ref[:]
    m = jnp.mean(x, axis=1, keepdims=True)
    v = jnp.mean((x - m) * (x - m), axis=1, keepdims=True)
    x = g1_ref[:] * (x - m) / jnp.sqrt(v + eps) + bb1_ref[:]
    x = jnp.maximum(x, 0.0)
    x = jnp.dot(w2_ref[:], x, preferred_element_type=jnp.float32) + b2_ref[:]
    m = jnp.mean(x, axis=1, keepdims=True)
    v = jnp.mean((x - m) * (x - m), axis=1, keepdims=True)
    x = g2_ref[:] * (x - m) / jnp.sqrt(v + eps) + bb2_ref[:]
    x = jnp.maximum(x, 0.0)
    o_ref[:] = jnp.dot(w3_ref[:], x, preferred_element_type=jnp.float32) + b3_ref[:]


def kernel(M, H, edge_index, params):
    N = M.shape[0]
    E = H.shape[0]
    src = edge_index[0].astype(jnp.int32)
    dst = edge_index[1].astype(jnp.int32)
    p = params

    gather_call = _make_gather()

    for _ in range(NLAYER - 1):
        mi_f, mj_f = gather_call(M.reshape(-1), dst, src)
        M_i = mi_f.reshape(E, 3)
        M_j = mj_f.reshape(E, 3)
        H = _mlp3(jnp.concatenate([M_i, M_j, H], axis=1), p, 'e')
        M_msg = _mlp3(jnp.concatenate([M_i, H], axis=1), p, 'v')
        M = jax.ops.segment_sum(M_msg, dst, num_segments=N)
    mi_f, mj_f = gather_call(M.reshape(-1), dst, src)
    M_i = mi_f.reshape(E, 3)
    M_j = mj_f.reshape(E, 3)
    H = _mlp3(jnp.concatenate([M_i, M_j, H], axis=1), p, 'e')

    c2 = lambda a: a.reshape(-1, 1)
    out_t = pl.pallas_call(
        _mlp_o_body,
        out_shape=jax.ShapeDtypeStruct((2, E), jnp.float32),
    )(H.T, p['o_l1_W'], c2(p['o_l1_b']), c2(p['o_bn1_g']), c2(p['o_bn1_b']),
      p['o_l2_W'], c2(p['o_l2_b']), c2(p['o_bn2_g']), c2(p['o_bn2_b']),
      p['o_l3_W'], c2(p['o_l3_b']))
    return out_t.T
